# baseline (speedup 1.0000x reference)
import jax
import jax.numpy as jnp
from jax.experimental import pallas as pl
from jax.experimental.pallas import tpu as pltpu

_C = 30
_BB = 128


def _iou_rows(pb, lb):
    pcx, pcy, pw, ph = pb
    lcx, lcy, lw, lh = lb
    p_l = pcx - 0.5 * pw
    p_r = pcx + 0.5 * pw
    p_t = pcy - 0.5 * ph
    p_b = pcy + 0.5 * ph
    l_l = lcx - 0.5 * lw
    l_r = lcx + 0.5 * lw
    l_t = lcy - 0.5 * lh
    l_b = lcy + 0.5 * lh
    mxl = jnp.maximum(p_l, l_l)
    mnr = jnp.minimum(p_r, l_r)
    mxt = jnp.maximum(p_t, l_t)
    mnb = jnp.minimum(p_b, l_b)
    inter = (mnr - mxl) * (mnb - mxt)
    ov = (mxl < mnr) & (mxt < mnb)
    area_p = pw * ph
    area_l = lw * lh
    return jnp.where(ov, inter / area_p + area_l - inter, 0.0)


def _to_channel_major(x):
    cells = jnp.stack([x[:, _C * j : _C * (j + 1)] for j in range(49)], axis=0)
    return jnp.transpose(cells, (2, 0, 1))


def _loss_body(p_ref, l_ref, o_ref):
    pt = _to_channel_major(p_ref[...])
    lt = _to_channel_major(l_ref[...])

    iou1 = _iou_rows((pt[0], pt[1], pt[2], pt[3]), (lt[0], lt[1], lt[2], lt[3]))
    iou2 = _iou_rows((pt[5], pt[6], pt[7], pt[8]), (lt[5], lt[6], lt[7], lt[8]))
    resp = iou1 > iou2

    xy1 = (pt[0] - lt[0]) ** 2 + (pt[1] - lt[1]) ** 2
    xy2 = (pt[5] - lt[5]) ** 2 + (pt[6] - lt[6]) ** 2
    wh1 = (jnp.sqrt(pt[2]) - jnp.sqrt(lt[2])) ** 2 + (jnp.sqrt(pt[3]) - jnp.sqrt(lt[3])) ** 2
    wh2 = (jnp.sqrt(pt[7]) - jnp.sqrt(lt[7])) ** 2 + (jnp.sqrt(pt[8]) - jnp.sqrt(lt[8])) ** 2

    t1 = (pt[4] - iou1) ** 2
    t2 = (pt[9] - iou2) ** 2
    conf_pair = jnp.where(resp, t1 + 0.5 * t2, t2 + 0.5 * t1)

    dcls = pt[10:] - lt[10:]
    cls = jnp.sum(dcls * dcls, axis=0)

    obj_cell = (
        5.0 * jnp.where(resp, xy1, xy2)
        + jnp.where(resp, wh1, wh2)
        + conf_pair
        + cls
    )
    noobj_cell = 0.5 * (pt[4] * pt[4] + pt[9] * pt[9])

    cell = jnp.where(lt[4] == 1.0, obj_cell, noobj_cell)
    o_ref[...] = jnp.sum(cell, axis=0, keepdims=True)[None].astype(o_ref.dtype)


@jax.jit
def kernel(preds, labels):
    b = preds.shape[0]
    p2 = preds.reshape(b, 7 * 7 * _C)
    l2 = labels.reshape(b, 7 * 7 * _C)
    g = b // _BB

    partials = pl.pallas_call(
        _loss_body,
        grid=(g,),
        in_specs=[
            pl.BlockSpec((_BB, 7 * 7 * _C), lambda i: (i, 0)),
            pl.BlockSpec((_BB, 7 * 7 * _C), lambda i: (i, 0)),
        ],
        out_specs=pl.BlockSpec((1, 1, _BB), lambda i: (i, 0, 0)),
        out_shape=jax.ShapeDtypeStruct((g, 1, _BB), jnp.float32),
        compiler_params=pltpu.CompilerParams(
            dimension_semantics=("parallel",),
        ),
    )(p2, l2)

    return jnp.sum(partials) / b


# speedup vs baseline: 1.0401x; 1.0401x over previous
import jax
import jax.numpy as jnp
import numpy as np
from jax.experimental import pallas as pl
from jax.experimental.pallas import tpu as pltpu

_C = 30
_BB = 128
_NCELL = 49
_NB = 10


def _cls_mask() -> np.ndarray:
    m = np.zeros((7 * 7 * _C, 128), dtype=np.float32)
    for j in range(_NCELL):
        m[j * _C + 10 : (j + 1) * _C, j] = 1.0
    return m


def _iou_rows(pb, lb):
    pcx, pcy, pw, ph = pb
    lcx, lcy, lw, lh = lb
    p_l = pcx - 0.5 * pw
    p_r = pcx + 0.5 * pw
    p_t = pcy - 0.5 * ph
    p_b = pcy + 0.5 * ph
    l_l = lcx - 0.5 * lw
    l_r = lcx + 0.5 * lw
    l_t = lcy - 0.5 * lh
    l_b = lcy + 0.5 * lh
    mxl = jnp.maximum(p_l, l_l)
    mnr = jnp.minimum(p_r, l_r)
    mxt = jnp.maximum(p_t, l_t)
    mnb = jnp.minimum(p_b, l_b)
    inter = (mnr - mxl) * (mnb - mxt)
    ov = (mxl < mnr) & (mxt < mnb)
    area_p = pw * ph
    area_l = lw * lh
    return jnp.where(ov, inter / area_p + area_l - inter, 0.0)


def _to_channel_major(x):
    cells = jnp.stack([x[:, _C * j : _C * j + _NB] for j in range(_NCELL)], axis=0)
    return jnp.transpose(cells, (2, 0, 1))


def _loss_body(p_ref, l_ref, m_ref, o_ref):
    p = p_ref[...]
    l = l_ref[...]

    d = p - l
    cls_bj = jnp.dot(d * d, m_ref[...], preferred_element_type=jnp.float32)
    cls = jnp.transpose(cls_bj)[:_NCELL]

    pt = _to_channel_major(p)
    lt = _to_channel_major(l)

    iou1 = _iou_rows((pt[0], pt[1], pt[2], pt[3]), (lt[0], lt[1], lt[2], lt[3]))
    iou2 = _iou_rows((pt[5], pt[6], pt[7], pt[8]), (lt[5], lt[6], lt[7], lt[8]))
    resp = iou1 > iou2

    xy1 = (pt[0] - lt[0]) ** 2 + (pt[1] - lt[1]) ** 2
    xy2 = (pt[5] - lt[5]) ** 2 + (pt[6] - lt[6]) ** 2
    wh1 = (jnp.sqrt(pt[2]) - jnp.sqrt(lt[2])) ** 2 + (jnp.sqrt(pt[3]) - jnp.sqrt(lt[3])) ** 2
    wh2 = (jnp.sqrt(pt[7]) - jnp.sqrt(lt[7])) ** 2 + (jnp.sqrt(pt[8]) - jnp.sqrt(lt[8])) ** 2

    t1 = (pt[4] - iou1) ** 2
    t2 = (pt[9] - iou2) ** 2
    conf_pair = jnp.where(resp, t1 + 0.5 * t2, t2 + 0.5 * t1)

    obj_cell = (
        5.0 * jnp.where(resp, xy1, xy2)
        + jnp.where(resp, wh1, wh2)
        + conf_pair
        + cls
    )
    noobj_cell = 0.5 * (pt[4] * pt[4] + pt[9] * pt[9])

    cell = jnp.where(lt[4] == 1.0, obj_cell, noobj_cell)
    o_ref[...] = jnp.sum(cell, axis=0, keepdims=True)[None].astype(o_ref.dtype)


@jax.jit
def kernel(preds, labels):
    b = preds.shape[0]
    p2 = preds.reshape(b, 7 * 7 * _C)
    l2 = labels.reshape(b, 7 * 7 * _C)
    m = jnp.asarray(_cls_mask())
    g = b // _BB

    partials = pl.pallas_call(
        _loss_body,
        grid=(g,),
        in_specs=[
            pl.BlockSpec((_BB, 7 * 7 * _C), lambda i: (i, 0)),
            pl.BlockSpec((_BB, 7 * 7 * _C), lambda i: (i, 0)),
            pl.BlockSpec((7 * 7 * _C, 128), lambda i: (0, 0)),
        ],
        out_specs=pl.BlockSpec((1, 1, _BB), lambda i: (i, 0, 0)),
        out_shape=jax.ShapeDtypeStruct((g, 1, _BB), jnp.float32),
        compiler_params=pltpu.CompilerParams(
            dimension_semantics=("parallel",),
        ),
    )(p2, l2, m)

    return jnp.sum(partials) / b


# speedup vs baseline: 1.0495x; 1.0090x over previous
import jax
import jax.numpy as jnp
import numpy as np
from jax.experimental import pallas as pl
from jax.experimental.pallas import tpu as pltpu

_C = 30
_BB = 256
_NCELL = 49
_NB = 10


def _cls_mask() -> np.ndarray:
    m = np.zeros((7 * 7 * _C, 128), dtype=np.float32)
    for j in range(_NCELL):
        m[j * _C + 10 : (j + 1) * _C, j] = 1.0
    return m


def _iou_rows(pb, lb):
    pcx, pcy, pw, ph = pb
    lcx, lcy, lw, lh = lb
    p_l = pcx - 0.5 * pw
    p_r = pcx + 0.5 * pw
    p_t = pcy - 0.5 * ph
    p_b = pcy + 0.5 * ph
    l_l = lcx - 0.5 * lw
    l_r = lcx + 0.5 * lw
    l_t = lcy - 0.5 * lh
    l_b = lcy + 0.5 * lh
    mxl = jnp.maximum(p_l, l_l)
    mnr = jnp.minimum(p_r, l_r)
    mxt = jnp.maximum(p_t, l_t)
    mnb = jnp.minimum(p_b, l_b)
    inter = (mnr - mxl) * (mnb - mxt)
    ov = (mxl < mnr) & (mxt < mnb)
    area_p = pw * ph
    area_l = lw * lh
    return jnp.where(ov, inter / area_p + area_l - inter, 0.0)


def _to_channel_major(x):
    cells = jnp.stack([x[:, _C * j : _C * j + _NB] for j in range(_NCELL)], axis=0)
    return jnp.transpose(cells, (2, 0, 1))


def _loss_body(p_ref, l_ref, m_ref, o_ref):
    p = p_ref[...]
    l = l_ref[...]

    d = p - l
    cls_bj = jnp.dot(d * d, m_ref[...], preferred_element_type=jnp.float32)
    cls = jnp.transpose(cls_bj)[:_NCELL]

    pt = _to_channel_major(p)
    lt = _to_channel_major(l)

    iou1 = _iou_rows((pt[0], pt[1], pt[2], pt[3]), (lt[0], lt[1], lt[2], lt[3]))
    iou2 = _iou_rows((pt[5], pt[6], pt[7], pt[8]), (lt[5], lt[6], lt[7], lt[8]))
    resp = iou1 > iou2

    xy1 = (pt[0] - lt[0]) ** 2 + (pt[1] - lt[1]) ** 2
    xy2 = (pt[5] - lt[5]) ** 2 + (pt[6] - lt[6]) ** 2
    wh1 = (jnp.sqrt(pt[2]) - jnp.sqrt(lt[2])) ** 2 + (jnp.sqrt(pt[3]) - jnp.sqrt(lt[3])) ** 2
    wh2 = (jnp.sqrt(pt[7]) - jnp.sqrt(lt[7])) ** 2 + (jnp.sqrt(pt[8]) - jnp.sqrt(lt[8])) ** 2

    t1 = (pt[4] - iou1) ** 2
    t2 = (pt[9] - iou2) ** 2
    conf_pair = jnp.where(resp, t1 + 0.5 * t2, t2 + 0.5 * t1)

    obj_cell = (
        5.0 * jnp.where(resp, xy1, xy2)
        + jnp.where(resp, wh1, wh2)
        + conf_pair
        + cls
    )
    noobj_cell = 0.5 * (pt[4] * pt[4] + pt[9] * pt[9])

    cell = jnp.where(lt[4] == 1.0, obj_cell, noobj_cell)
    o_ref[...] = jnp.sum(cell, axis=0, keepdims=True)[None].astype(o_ref.dtype)


@jax.jit
def kernel(preds, labels):
    b = preds.shape[0]
    p2 = preds.reshape(b, 7 * 7 * _C)
    l2 = labels.reshape(b, 7 * 7 * _C)
    m = jnp.asarray(_cls_mask())
    g = b // _BB

    partials = pl.pallas_call(
        _loss_body,
        grid=(g,),
        in_specs=[
            pl.BlockSpec((_BB, 7 * 7 * _C), lambda i: (i, 0)),
            pl.BlockSpec((_BB, 7 * 7 * _C), lambda i: (i, 0)),
            pl.BlockSpec((7 * 7 * _C, 128), lambda i: (0, 0)),
        ],
        out_specs=pl.BlockSpec((1, 1, _BB), lambda i: (i, 0, 0)),
        out_shape=jax.ShapeDtypeStruct((g, 1, _BB), jnp.float32),
        compiler_params=pltpu.CompilerParams(
            dimension_semantics=("parallel",),
        ),
    )(p2, l2, m)

    return jnp.sum(partials) / b


# speedup vs baseline: 1.0551x; 1.0053x over previous
import jax
import jax.numpy as jnp
import numpy as np
from jax.experimental import pallas as pl
from jax.experimental.pallas import tpu as pltpu

_C = 30
_BB = 512
_NCELL = 49
_NB = 10


def _cls_mask() -> np.ndarray:
    m = np.zeros((7 * 7 * _C, 128), dtype=np.float32)
    for j in range(_NCELL):
        m[j * _C + 10 : (j + 1) * _C, j] = 1.0
    return m


def _iou_rows(pb, lb):
    pcx, pcy, pw, ph = pb
    lcx, lcy, lw, lh = lb
    p_l = pcx - 0.5 * pw
    p_r = pcx + 0.5 * pw
    p_t = pcy - 0.5 * ph
    p_b = pcy + 0.5 * ph
    l_l = lcx - 0.5 * lw
    l_r = lcx + 0.5 * lw
    l_t = lcy - 0.5 * lh
    l_b = lcy + 0.5 * lh
    mxl = jnp.maximum(p_l, l_l)
    mnr = jnp.minimum(p_r, l_r)
    mxt = jnp.maximum(p_t, l_t)
    mnb = jnp.minimum(p_b, l_b)
    inter = (mnr - mxl) * (mnb - mxt)
    ov = (mxl < mnr) & (mxt < mnb)
    area_p = pw * ph
    area_l = lw * lh
    return jnp.where(ov, inter / area_p + area_l - inter, 0.0)


def _to_channel_major(x):
    cells = jnp.stack([x[:, _C * j : _C * j + _NB] for j in range(_NCELL)], axis=0)
    return jnp.transpose(cells, (2, 0, 1))


def _loss_body(p_ref, l_ref, m_ref, o_ref):
    p = p_ref[...]
    l = l_ref[...]

    d = p - l
    cls_bj = jnp.dot(d * d, m_ref[...], preferred_element_type=jnp.float32)
    cls = jnp.transpose(cls_bj)[:_NCELL]

    pt = _to_channel_major(p)
    lt = _to_channel_major(l)

    iou1 = _iou_rows((pt[0], pt[1], pt[2], pt[3]), (lt[0], lt[1], lt[2], lt[3]))
    iou2 = _iou_rows((pt[5], pt[6], pt[7], pt[8]), (lt[5], lt[6], lt[7], lt[8]))
    resp = iou1 > iou2

    xy1 = (pt[0] - lt[0]) ** 2 + (pt[1] - lt[1]) ** 2
    xy2 = (pt[5] - lt[5]) ** 2 + (pt[6] - lt[6]) ** 2
    wh1 = (jnp.sqrt(pt[2]) - jnp.sqrt(lt[2])) ** 2 + (jnp.sqrt(pt[3]) - jnp.sqrt(lt[3])) ** 2
    wh2 = (jnp.sqrt(pt[7]) - jnp.sqrt(lt[7])) ** 2 + (jnp.sqrt(pt[8]) - jnp.sqrt(lt[8])) ** 2

    t1 = (pt[4] - iou1) ** 2
    t2 = (pt[9] - iou2) ** 2
    conf_pair = jnp.where(resp, t1 + 0.5 * t2, t2 + 0.5 * t1)

    obj_cell = (
        5.0 * jnp.where(resp, xy1, xy2)
        + jnp.where(resp, wh1, wh2)
        + conf_pair
        + cls
    )
    noobj_cell = 0.5 * (pt[4] * pt[4] + pt[9] * pt[9])

    cell = jnp.where(lt[4] == 1.0, obj_cell, noobj_cell)
    o_ref[...] = jnp.sum(cell, axis=0, keepdims=True)[None].astype(o_ref.dtype)


@jax.jit
def kernel(preds, labels):
    b = preds.shape[0]
    p2 = preds.reshape(b, 7 * 7 * _C)
    l2 = labels.reshape(b, 7 * 7 * _C)
    m = jnp.asarray(_cls_mask())
    g = b // _BB

    partials = pl.pallas_call(
        _loss_body,
        grid=(g,),
        in_specs=[
            pl.BlockSpec((_BB, 7 * 7 * _C), lambda i: (i, 0)),
            pl.BlockSpec((_BB, 7 * 7 * _C), lambda i: (i, 0)),
            pl.BlockSpec((7 * 7 * _C, 128), lambda i: (0, 0)),
        ],
        out_specs=pl.BlockSpec((1, 1, _BB), lambda i: (i, 0, 0)),
        out_shape=jax.ShapeDtypeStruct((g, 1, _BB), jnp.float32),
        compiler_params=pltpu.CompilerParams(
            dimension_semantics=("parallel",),
        ),
    )(p2, l2, m)

    return jnp.sum(partials) / b
